# Initial kernel scaffold; baseline (speedup 1.0000x reference)
import jax
import jax.numpy as jnp
from jax.experimental import pallas as pl


def kernel(input_tensor, pos_table):
    raise NotImplementedError("write your pallas kernel here")


# speedup vs baseline: 3.6852x; 3.6852x over previous
import jax
import jax.numpy as jnp
from jax.experimental import pallas as pl


def _add_body(x_ref, t_ref, o_ref):
    o_ref[...] = x_ref[...] + t_ref[...][None, :, :]


def kernel(input_tensor, pos_table):
    B, S, E = input_tensor.shape
    BB = 32
    return pl.pallas_call(
        _add_body,
        grid=(B // BB,),
        in_specs=[
            pl.BlockSpec((BB, S, E), lambda i: (i, 0, 0)),
            pl.BlockSpec((S, E), lambda i: (0, 0)),
        ],
        out_specs=pl.BlockSpec((BB, S, E), lambda i: (i, 0, 0)),
        out_shape=jax.ShapeDtypeStruct((B, S, E), input_tensor.dtype),
    )(input_tensor, pos_table)


# speedup vs baseline: 6.1017x; 1.6557x over previous
import jax
import jax.numpy as jnp
from jax.experimental import pallas as pl


def _add_body(x_ref, t_ref, o_ref):
    o_ref[...] = x_ref[...] + t_ref[...]


def kernel(input_tensor, pos_table):
    B, S, E = input_tensor.shape
    D = S * E
    x = input_tensor.reshape(B, D)
    t = pos_table.reshape(1, D)
    BB = 64
    out = pl.pallas_call(
        _add_body,
        grid=(B // BB,),
        in_specs=[
            pl.BlockSpec((BB, D), lambda i: (i, 0)),
            pl.BlockSpec((1, D), lambda i: (0, 0)),
        ],
        out_specs=pl.BlockSpec((BB, D), lambda i: (i, 0)),
        out_shape=jax.ShapeDtypeStruct((B, D), input_tensor.dtype),
    )(x, t)
    return out.reshape(B, S, E)


# speedup vs baseline: 6.1480x; 1.0076x over previous
import jax
import jax.numpy as jnp
from jax.experimental import pallas as pl


def _add_body(x_ref, t_ref, o_ref):
    o_ref[...] = x_ref[...] + t_ref[...]


def kernel(input_tensor, pos_table):
    B, S, E = input_tensor.shape
    D = S * E
    x = input_tensor.reshape(B, D)
    t = pos_table.reshape(1, D)
    BB = 128
    out = pl.pallas_call(
        _add_body,
        grid=(B // BB,),
        in_specs=[
            pl.BlockSpec((BB, D), lambda i: (i, 0)),
            pl.BlockSpec((1, D), lambda i: (0, 0)),
        ],
        out_specs=pl.BlockSpec((BB, D), lambda i: (i, 0)),
        out_shape=jax.ShapeDtypeStruct((B, D), input_tensor.dtype),
    )(x, t)
    return out.reshape(B, S, E)
